# baseline (speedup 1.0000x reference)
import functools

import jax
import jax.numpy as jnp
from jax import lax
from jax.experimental import pallas as pl
from jax.experimental.pallas import tpu as pltpu
from jax.experimental.pallas import tpu_sc as plsc

NC = 2
NS = 16
NW = NC * NS
L = 16

P = 6_400_000
SLOTS_PER_W = P // NW
CHUNK = 4000
N_CHUNKS = SLOTS_PER_W // CHUNK
VECS = CHUNK // L
NG = (VECS + L - 1) // L
CHUNK_T = NG * L * L
CUMS_PER_W = N_CHUNKS * CHUNK_T
STAGE = 17
VO_BATCH = 4
VO_PER_W = 12504

NNETS = 100_000

NODES_PER_W = 3200
NPAD = NW * NODES_PER_W
NODE_CHUNK = 1600
SPAD_LEN = (NW - 1) * NODES_PER_W + NODE_CHUNK + 1608

_mesh = plsc.VectorSubcoreMesh(core_axis_name="c", subcore_axis_name="s")
_params = pltpu.CompilerParams(needs_layout_passes=False)


def _wid():
    return lax.axis_index("s") * NC + lax.axis_index("c")


@functools.partial(
    pl.kernel,
    out_type=(
        jax.ShapeDtypeStruct((NW * CUMS_PER_W,), jnp.float32),
        jax.ShapeDtypeStruct((NW * VO_PER_W,), jnp.float32),
        jax.ShapeDtypeStruct((NW, L), jnp.float32),
    ),
    mesh=_mesh,
    compiler_params=_params,
    scratch_types=(
        pltpu.VMEM((NNETS,), jnp.float32),
        pltpu.VMEM((CHUNK,), jnp.int32),
        pltpu.VMEM((CHUNK,), jnp.int32),
        pltpu.VMEM((CHUNK,), jnp.int32),
        pltpu.VMEM((CHUNK,), jnp.int32),
        pltpu.VMEM((NG * L * STAGE,), jnp.float32),
        pltpu.VMEM((CHUNK_T,), jnp.float32),
        pltpu.VMEM((CHUNK_T,), jnp.float32),
        pltpu.VMEM((1024,), jnp.float32),
        pltpu.VMEM((L,), jnp.float32),
        pltpu.SemaphoreType.DMA,
        pltpu.SemaphoreType.DMA,
        pltpu.SemaphoreType.DMA,
        pltpu.SemaphoreType.DMA,
    ),
)
def _phase1(wt_hbm, fnp_hbm, p2n_hbm, cums_hbm, vo_hbm, totals_hbm,
            wt_v, fnp0, fnp1, net0, net1, stage_v, cumsT0, cumsT1, vo_buf,
            tot_v, sem0, sem1, csem0, csem1):
    wid = _wid()
    base = wid * SLOTS_PER_W
    cums_base = wid * CUMS_PER_W
    vo_base = wid * VO_PER_W
    pltpu.sync_copy(wt_hbm, wt_v)

    fnp_b = (fnp0, fnp1)
    net_b = (net0, net1)
    sem_b = (sem0, sem1)
    cums_b = (cumsT0, cumsT1)
    csem_b = (csem0, csem1)
    lanes = lax.iota(jnp.int32, L)
    lanes17 = lanes * STAGE

    def _chunk_dst(k):
        off = pl.multiple_of(cums_base + k * CHUNK_T, 8)
        return cums_hbm.at[pl.ds(off, CHUNK_T)]

    def _prefetch(k, b):
        off = pl.multiple_of(base + k * CHUNK, 8)
        pltpu.sync_copy(fnp_hbm.at[pl.ds(off, CHUNK)], fnp_b[b])
        pltpu.async_copy(p2n_hbm.at[fnp_b[b]], net_b[b], sem_b[b])

    def _compute(k, b, cin):
        pltpu.make_async_copy(p2n_hbm.at[fnp_b[b]], net_b[b], sem_b[b]).wait()
        pltpu.make_async_copy(cums_b[b], _chunk_dst(k), csem_b[b]).wait()

        @plsc.parallel_loop(0, VECS, unroll=8)
        def _(j):
            idx = net_b[b][pl.ds(j * L, L)]
            stage_v[pl.ds(j * STAGE, L)] = plsc.load_gather(wt_v, [idx])

        def grp(g, cin):
            gbase = g * (L * STAGE)
            acc = plsc.load_gather(stage_v, [lanes17 + gbase])
            cums_b[b][pl.ds(g * 256, L)] = acc
            for c in range(1, L):
                acc = acc + plsc.load_gather(stage_v, [lanes17 + (gbase + c)])
                cums_b[b][pl.ds(g * 256 + c * L, L)] = acc
            vs = jnp.where(g * L + lanes < VECS, acc, jnp.float32(0.0))
            inc = plsc.cumsum(vs) + cin
            vo_buf[pl.ds((k % VO_BATCH) * VECS + g * L, L)] = inc - vs
            return inc[L - 1]

        cin = lax.fori_loop(0, NG, grp, cin)
        pltpu.async_copy(cums_b[b], _chunk_dst(k), csem_b[b])

        @pl.when(k % VO_BATCH == VO_BATCH - 1)
        def _():
            voff = pl.multiple_of(vo_base + (k // VO_BATCH) * (VO_BATCH * VECS), 8)
            pltpu.sync_copy(vo_buf.at[pl.ds(0, VO_BATCH * VECS)],
                            vo_hbm.at[pl.ds(voff, VO_BATCH * VECS)])

        return cin

    pltpu.async_copy(cumsT0, _chunk_dst(0), csem0)
    pltpu.async_copy(cumsT1, _chunk_dst(1), csem1)
    _prefetch(0, 0)

    def body(c2, cin):
        k0 = 2 * c2
        _prefetch(k0 + 1, 1)
        cin = _compute(k0, 0, cin)

        @pl.when(c2 < N_CHUNKS // 2 - 1)
        def _():
            _prefetch(k0 + 2, 0)

        cin = _compute(k0 + 1, 1, cin)
        return cin

    total = lax.fori_loop(0, N_CHUNKS // 2, body, jnp.float32(0.0))
    pltpu.make_async_copy(cumsT0, _chunk_dst(N_CHUNKS - 2), csem0).wait()
    pltpu.make_async_copy(cumsT1, _chunk_dst(N_CHUNKS - 1), csem1).wait()
    n_tail = (N_CHUNKS // VO_BATCH) * (VO_BATCH * VECS)
    pltpu.sync_copy(vo_buf.at[pl.ds(0, VO_PER_W - n_tail)],
                    vo_hbm.at[pl.ds(pl.multiple_of(vo_base + n_tail, 8),
                                    VO_PER_W - n_tail)])
    tot_v[...] = jnp.full((L,), total, dtype=jnp.float32)
    pltpu.sync_copy(tot_v, totals_hbm.at[wid])


def _taddr(sc):
    wrk = sc // SLOTS_PER_W
    s = sc - wrk * SLOTS_PER_W
    ch = s // CHUNK
    rem = s - ch * CHUNK
    j = rem // L
    c = rem - j * L
    g = j // L
    r = j - g * L
    return wrk * CUMS_PER_W + ch * CHUNK_T + g * 256 + c * L + r


@functools.partial(
    pl.kernel,
    out_type=jax.ShapeDtypeStruct((NPAD,), jnp.float32),
    mesh=_mesh,
    compiler_params=_params,
    scratch_types=(
        pltpu.VMEM((NW,), jnp.float32),
        pltpu.VMEM((NW,), jnp.float32),
        pltpu.VMEM((1608,), jnp.int32),
        pltpu.VMEM((NODE_CHUNK,), jnp.int32),
        pltpu.VMEM((NODE_CHUNK,), jnp.int32),
        pltpu.VMEM((NODE_CHUNK,), jnp.int32),
        pltpu.VMEM((NODE_CHUNK,), jnp.int32),
        pltpu.VMEM((NODE_CHUNK,), jnp.float32),
        pltpu.VMEM((NODE_CHUNK,), jnp.float32),
        pltpu.VMEM((NODE_CHUNK,), jnp.float32),
        pltpu.VMEM((NODE_CHUNK,), jnp.float32),
        pltpu.VMEM((NODE_CHUNK,), jnp.float32),
        pltpu.SemaphoreType.DMA,
    ),
)
def _phase2(cums_hbm, vo_hbm, tot32_hbm, start_hbm, out_hbm,
            tot_v, bases_v, sv, eidx, bidx, evo, bvo,
            le_v, lb_v, ve_v, vb_v, ov, sem):
    wid = _wid()
    pltpu.sync_copy(tot32_hbm, tot_v)
    t0 = tot_v[pl.ds(0, L)]
    t1 = tot_v[pl.ds(L, L)]
    bases_v[pl.ds(0, L)] = plsc.cumsum(t0) - t0
    bases_v[pl.ds(L, L)] = plsc.cumsum(t1) - t1 + jnp.sum(t0)

    def chunk_body(c, _):
        n0 = pl.multiple_of(wid * NODES_PER_W + c * NODE_CHUNK, 8)
        pltpu.sync_copy(start_hbm.at[pl.ds(n0, 1608)], sv)

        def build(j, _):
            lo = sv[pl.ds(j * L, L)]
            hi = sv[pl.ds(j * L + 1, L)]
            ec = jnp.maximum(hi - 1, 0)
            bc = jnp.maximum(lo - 1, 0)
            we = ec // SLOTS_PER_W
            wb = bc // SLOTS_PER_W
            eidx[pl.ds(j * L, L)] = _taddr(ec)
            bidx[pl.ds(j * L, L)] = _taddr(bc)
            evo[pl.ds(j * L, L)] = we * VO_PER_W + (ec - we * SLOTS_PER_W) // L
            bvo[pl.ds(j * L, L)] = wb * VO_PER_W + (bc - wb * SLOTS_PER_W) // L
            return 0

        lax.fori_loop(0, NODE_CHUNK // L, build, 0, unroll=4)
        pltpu.async_copy(cums_hbm.at[eidx], le_v, sem)
        pltpu.async_copy(cums_hbm.at[bidx], lb_v, sem)
        pltpu.async_copy(vo_hbm.at[evo], ve_v, sem)
        pltpu.async_copy(vo_hbm.at[bvo], vb_v, sem)
        pltpu.make_async_copy(cums_hbm.at[eidx], le_v, sem).wait()
        pltpu.make_async_copy(cums_hbm.at[bidx], lb_v, sem).wait()
        pltpu.make_async_copy(vo_hbm.at[evo], ve_v, sem).wait()
        pltpu.make_async_copy(vo_hbm.at[bvo], vb_v, sem).wait()

        def comp(j, _):
            lo = sv[pl.ds(j * L, L)]
            hi = sv[pl.ds(j * L + 1, L)]
            e = hi - 1
            b = lo - 1
            ec = jnp.maximum(e, 0)
            bc = jnp.maximum(b, 0)
            be = plsc.load_gather(bases_v, [ec // SLOTS_PER_W])
            bb = plsc.load_gather(bases_v, [bc // SLOTS_PER_W])
            ge = le_v[pl.ds(j * L, L)] + ve_v[pl.ds(j * L, L)] + be
            gb = lb_v[pl.ds(j * L, L)] + vb_v[pl.ds(j * L, L)] + bb
            ge = jnp.where(e < 0, jnp.float32(0.0), ge)
            gb = jnp.where(b < 0, jnp.float32(0.0), gb)
            ov[pl.ds(j * L, L)] = ge - gb
            return 0

        lax.fori_loop(0, NODE_CHUNK // L, comp, 0, unroll=4)
        pltpu.sync_copy(ov, out_hbm.at[pl.ds(n0, NODE_CHUNK)])
        return 0

    lax.fori_loop(0, NODES_PER_W // NODE_CHUNK, chunk_body, 0)


def kernel(net_weights, flat_nodepin, nodepin_start, pin2net_map, num_nodes):
    n = nodepin_start.shape[0] - 1
    p = flat_nodepin.shape[0]
    fnp = flat_nodepin.astype(jnp.int32)
    p2n = pin2net_map.astype(jnp.int32)
    start_pad = jnp.concatenate([
        nodepin_start.astype(jnp.int32),
        jnp.full((SPAD_LEN - (n + 1),), p, dtype=jnp.int32),
    ])
    cums, vo, totals = _phase1(net_weights, fnp, p2n)
    out_pad = _phase2(cums, vo, totals[:, 0], start_pad)
    return out_pad[:n]


# speedup vs baseline: 1.0147x; 1.0147x over previous
import functools

import jax
import jax.numpy as jnp
from jax import lax
from jax.experimental import pallas as pl
from jax.experimental.pallas import tpu as pltpu
from jax.experimental.pallas import tpu_sc as plsc

NC = 2
NS = 16
NW = NC * NS
L = 16

P = 6_400_000
SLOTS_PER_W = P // NW
CHUNK = 4000
N_CHUNKS = SLOTS_PER_W // CHUNK

NNETS = 100_000

NODES_PER_W = 3200
NPAD = NW * NODES_PER_W
SV_WIN = NODES_PER_W + 8
SPAD_LEN = NPAD + 8

_mesh = plsc.VectorSubcoreMesh(core_axis_name="c", subcore_axis_name="s")
_params = pltpu.CompilerParams(needs_layout_passes=False)


def _wid():
    return lax.axis_index("s") * NC + lax.axis_index("c")


@functools.partial(
    pl.kernel,
    out_type=(
        jax.ShapeDtypeStruct((P,), jnp.float32),
        jax.ShapeDtypeStruct((NW, L), jnp.float32),
    ),
    mesh=_mesh,
    compiler_params=_params,
    scratch_types=(
        pltpu.VMEM((NNETS,), jnp.float32),
        pltpu.VMEM((CHUNK,), jnp.int32),
        pltpu.VMEM((CHUNK,), jnp.int32),
        pltpu.VMEM((CHUNK,), jnp.int32),
        pltpu.VMEM((CHUNK,), jnp.int32),
        pltpu.VMEM((CHUNK,), jnp.float32),
        pltpu.VMEM((CHUNK,), jnp.float32),
        pltpu.VMEM((L,), jnp.float32),
        pltpu.SemaphoreType.DMA,
        pltpu.SemaphoreType.DMA,
        pltpu.SemaphoreType.DMA,
        pltpu.SemaphoreType.DMA,
        pltpu.SemaphoreType.DMA,
        pltpu.SemaphoreType.DMA,
    ),
)
def _phase1(wt_hbm, fnp_hbm, p2n_hbm, cums_hbm, totals_hbm,
            wt_v, fnp0, fnp1, net0, net1, cums0, cums1, tot_v,
            sem0, sem1, csem0, csem1, fsem0, fsem1):
    wid = _wid()
    base = wid * SLOTS_PER_W
    pltpu.sync_copy(wt_hbm, wt_v)

    fnp_b = (fnp0, fnp1)
    net_b = (net0, net1)
    sem_b = (sem0, sem1)
    cums_b = (cums0, cums1)
    csem_b = (csem0, csem1)
    fsem_b = (fsem0, fsem1)

    def _chunk_dst(k):
        off = pl.multiple_of(base + k * CHUNK, 8)
        return cums_hbm.at[pl.ds(off, CHUNK)]

    def _fnp_src(k):
        off = pl.multiple_of(base + k * CHUNK, 8)
        return fnp_hbm.at[pl.ds(off, CHUNK)]

    def _step(k, b, carry):
        pltpu.make_async_copy(p2n_hbm.at[fnp_b[b]], net_b[b], sem_b[b]).wait()

        @pl.when(k < N_CHUNKS - 2)
        def _():
            pltpu.async_copy(_fnp_src(k + 2), fnp_b[b], fsem_b[b])

        @pl.when(k < N_CHUNKS - 1)
        def _():
            pltpu.make_async_copy(_fnp_src(k + 1), fnp_b[1 - b],
                                  fsem_b[1 - b]).wait()
            pltpu.async_copy(p2n_hbm.at[fnp_b[1 - b]], net_b[1 - b],
                             sem_b[1 - b])

        pltpu.make_async_copy(cums_b[b], _chunk_dst(k), csem_b[b]).wait()

        def vec_body(j, cin):
            idx = net_b[b][pl.ds(j * L, L)]
            w = plsc.load_gather(wt_v, [idx])
            cs = plsc.cumsum(w) + cin
            cums_b[b][pl.ds(j * L, L)] = cs
            return cs[L - 1]

        carry = lax.fori_loop(0, CHUNK // L, vec_body, carry, unroll=4)
        pltpu.async_copy(cums_b[b], _chunk_dst(k), csem_b[b])
        return carry

    pltpu.async_copy(cums0, _chunk_dst(0), csem0)
    pltpu.async_copy(cums1, _chunk_dst(1), csem1)
    pltpu.async_copy(_fnp_src(0), fnp0, fsem0)
    pltpu.async_copy(_fnp_src(1), fnp1, fsem1)
    pltpu.make_async_copy(_fnp_src(0), fnp0, fsem0).wait()
    pltpu.async_copy(p2n_hbm.at[fnp0], net0, sem0)

    def body(c2, carry):
        k0 = 2 * c2
        carry = _step(k0, 0, carry)
        carry = _step(k0 + 1, 1, carry)
        return carry

    total = lax.fori_loop(0, N_CHUNKS // 2, body, jnp.float32(0.0))
    pltpu.make_async_copy(cums0, _chunk_dst(N_CHUNKS - 2), csem0).wait()
    pltpu.make_async_copy(cums1, _chunk_dst(N_CHUNKS - 1), csem1).wait()
    tot_v[...] = jnp.full((L,), total, dtype=jnp.float32)
    pltpu.sync_copy(tot_v, totals_hbm.at[wid])


@functools.partial(
    pl.kernel,
    out_type=jax.ShapeDtypeStruct((NPAD,), jnp.float32),
    mesh=_mesh,
    compiler_params=_params,
    scratch_types=(
        pltpu.VMEM((NW,), jnp.float32),
        pltpu.VMEM((NW,), jnp.float32),
        pltpu.VMEM((SV_WIN,), jnp.int32),
        pltpu.VMEM((NODES_PER_W,), jnp.int32),
        pltpu.VMEM((NODES_PER_W,), jnp.int32),
        pltpu.VMEM((NODES_PER_W,), jnp.float32),
        pltpu.VMEM((NODES_PER_W,), jnp.float32),
        pltpu.VMEM((NODES_PER_W,), jnp.float32),
        pltpu.SemaphoreType.DMA,
    ),
)
def _phase2(cums_hbm, tot32_hbm, start_hbm, out_hbm,
            tot_v, bases_v, sv, eidx, bidx, le_v, lb_v, ov, sem):
    wid = _wid()
    pltpu.sync_copy(tot32_hbm, tot_v)
    t0 = tot_v[pl.ds(0, L)]
    t1 = tot_v[pl.ds(L, L)]
    bases_v[pl.ds(0, L)] = plsc.cumsum(t0) - t0
    bases_v[pl.ds(L, L)] = plsc.cumsum(t1) - t1 + jnp.sum(t0)

    n0 = pl.multiple_of(wid * NODES_PER_W, 8)
    pltpu.sync_copy(start_hbm.at[pl.ds(n0, SV_WIN)], sv)

    def build(j, _):
        lo = sv[pl.ds(j * L, L)]
        hi = sv[pl.ds(j * L + 1, L)]
        eidx[pl.ds(j * L, L)] = jnp.maximum(hi - 1, 0)
        bidx[pl.ds(j * L, L)] = jnp.maximum(lo - 1, 0)
        return 0

    lax.fori_loop(0, NODES_PER_W // L, build, 0, unroll=4)
    pltpu.async_copy(cums_hbm.at[eidx], le_v, sem)
    pltpu.async_copy(cums_hbm.at[bidx], lb_v, sem)
    pltpu.make_async_copy(cums_hbm.at[eidx], le_v, sem).wait()
    pltpu.make_async_copy(cums_hbm.at[bidx], lb_v, sem).wait()

    def comp(j, _):
        lo = sv[pl.ds(j * L, L)]
        hi = sv[pl.ds(j * L + 1, L)]
        e = hi - 1
        b = lo - 1
        ec = jnp.maximum(e, 0)
        bc = jnp.maximum(b, 0)
        be = plsc.load_gather(bases_v, [ec // SLOTS_PER_W])
        bb = plsc.load_gather(bases_v, [bc // SLOTS_PER_W])
        ge = jnp.where(e < 0, jnp.float32(0.0), le_v[pl.ds(j * L, L)] + be)
        gb = jnp.where(b < 0, jnp.float32(0.0), lb_v[pl.ds(j * L, L)] + bb)
        ov[pl.ds(j * L, L)] = ge - gb
        return 0

    lax.fori_loop(0, NODES_PER_W // L, comp, 0, unroll=4)
    pltpu.sync_copy(ov, out_hbm.at[pl.ds(n0, NODES_PER_W)])


def kernel(net_weights, flat_nodepin, nodepin_start, pin2net_map, num_nodes):
    n = nodepin_start.shape[0] - 1
    p = flat_nodepin.shape[0]
    fnp = flat_nodepin.astype(jnp.int32)
    p2n = pin2net_map.astype(jnp.int32)
    start_pad = jnp.concatenate([
        nodepin_start.astype(jnp.int32),
        jnp.full((SPAD_LEN - (n + 1),), p, dtype=jnp.int32),
    ])
    cums, totals = _phase1(net_weights, fnp, p2n)
    out_pad = _phase2(cums, totals[:, 0], start_pad)
    return out_pad[:n]


# speedup vs baseline: 1.0513x; 1.0361x over previous
import functools

import jax
import jax.numpy as jnp
from jax import lax
from jax.experimental import pallas as pl
from jax.experimental.pallas import tpu as pltpu
from jax.experimental.pallas import tpu_sc as plsc

NC = 2
NS = 16
NW = NC * NS
L = 16

P = 6_400_000
SLOTS_PER_W = P // NW
CHUNK = 4000
N_CHUNKS = SLOTS_PER_W // CHUNK

NNETS = 100_000

NODES_PER_W = 3200
NPAD = NW * NODES_PER_W
NODE_CHUNK = 1600
SPAD_LEN = (NW - 1) * NODES_PER_W + NODE_CHUNK + 1608

_mesh = plsc.VectorSubcoreMesh(core_axis_name="c", subcore_axis_name="s")
_params = pltpu.CompilerParams(needs_layout_passes=False)


def _wid():
    return lax.axis_index("s") * NC + lax.axis_index("c")


@functools.partial(
    pl.kernel,
    out_type=(
        jax.ShapeDtypeStruct((P,), jnp.float32),
        jax.ShapeDtypeStruct((NW, L), jnp.float32),
    ),
    mesh=_mesh,
    compiler_params=_params,
    scratch_types=(
        pltpu.VMEM((NNETS,), jnp.float32),
        pltpu.VMEM((CHUNK,), jnp.int32),
        pltpu.VMEM((CHUNK,), jnp.int32),
        pltpu.VMEM((CHUNK,), jnp.int32),
        pltpu.VMEM((CHUNK,), jnp.int32),
        pltpu.VMEM((CHUNK,), jnp.float32),
        pltpu.VMEM((CHUNK,), jnp.float32),
        pltpu.VMEM((L,), jnp.float32),
        pltpu.SemaphoreType.DMA,
        pltpu.SemaphoreType.DMA,
        pltpu.SemaphoreType.DMA,
        pltpu.SemaphoreType.DMA,
    ),
)
def _phase1(wt_hbm, fnp_hbm, p2n_hbm, cums_hbm, totals_hbm,
            wt_v, fnp0, fnp1, net0, net1, cums0, cums1, tot_v,
            sem0, sem1, csem0, csem1):
    wid = _wid()
    base = wid * SLOTS_PER_W
    pltpu.sync_copy(wt_hbm, wt_v)

    fnp_b = (fnp0, fnp1)
    net_b = (net0, net1)
    sem_b = (sem0, sem1)
    cums_b = (cums0, cums1)
    csem_b = (csem0, csem1)

    def _chunk_dst(k):
        off = pl.multiple_of(base + k * CHUNK, 8)
        return cums_hbm.at[pl.ds(off, CHUNK)]

    def _prefetch(k, b):
        off = pl.multiple_of(base + k * CHUNK, 8)
        pltpu.sync_copy(fnp_hbm.at[pl.ds(off, CHUNK)], fnp_b[b])
        pltpu.async_copy(p2n_hbm.at[fnp_b[b]], net_b[b], sem_b[b])

    def _compute(k, b, carry):
        pltpu.make_async_copy(p2n_hbm.at[fnp_b[b]], net_b[b], sem_b[b]).wait()
        pltpu.make_async_copy(cums_b[b], _chunk_dst(k), csem_b[b]).wait()

        def vec_body(j, cin):
            idx = net_b[b][pl.ds(j * L, L)]
            w = plsc.load_gather(wt_v, [idx])
            cs = plsc.cumsum(w) + cin
            cums_b[b][pl.ds(j * L, L)] = cs
            return cs[L - 1]

        carry = lax.fori_loop(0, CHUNK // L, vec_body, carry, unroll=4)
        pltpu.async_copy(cums_b[b], _chunk_dst(k), csem_b[b])
        return carry

    pltpu.async_copy(cums0, _chunk_dst(0), csem0)
    pltpu.async_copy(cums1, _chunk_dst(1), csem1)
    _prefetch(0, 0)

    def body(c2, carry):
        k0 = 2 * c2
        _prefetch(k0 + 1, 1)
        carry = _compute(k0, 0, carry)

        @pl.when(c2 < N_CHUNKS // 2 - 1)
        def _():
            _prefetch(k0 + 2, 0)

        carry = _compute(k0 + 1, 1, carry)
        return carry

    total = lax.fori_loop(0, N_CHUNKS // 2, body, jnp.float32(0.0))
    pltpu.make_async_copy(cums0, _chunk_dst(N_CHUNKS - 2), csem0).wait()
    pltpu.make_async_copy(cums1, _chunk_dst(N_CHUNKS - 1), csem1).wait()
    tot_v[...] = jnp.full((L,), total, dtype=jnp.float32)
    pltpu.sync_copy(tot_v, totals_hbm.at[wid])


@functools.partial(
    pl.kernel,
    out_type=jax.ShapeDtypeStruct((NPAD,), jnp.float32),
    mesh=_mesh,
    compiler_params=_params,
    scratch_types=(
        pltpu.VMEM((NW,), jnp.float32),
        pltpu.VMEM((NW,), jnp.float32),
        pltpu.VMEM((1608,), jnp.int32),
        pltpu.VMEM((NODE_CHUNK,), jnp.int32),
        pltpu.VMEM((NODE_CHUNK,), jnp.int32),
        pltpu.VMEM((NODE_CHUNK,), jnp.float32),
        pltpu.VMEM((NODE_CHUNK,), jnp.float32),
        pltpu.VMEM((NODE_CHUNK,), jnp.float32),
        pltpu.SemaphoreType.DMA,
    ),
)
def _phase2(cums_hbm, tot32_hbm, start_hbm, out_hbm,
            tot_v, bases_v, sv, eidx, bidx, le_v, lb_v, ov, sem):
    wid = _wid()
    pltpu.sync_copy(tot32_hbm, tot_v)
    t0 = tot_v[pl.ds(0, L)]
    t1 = tot_v[pl.ds(L, L)]
    bases_v[pl.ds(0, L)] = plsc.cumsum(t0) - t0
    bases_v[pl.ds(L, L)] = plsc.cumsum(t1) - t1 + jnp.sum(t0)

    def chunk_body(c, _):
        n0 = pl.multiple_of(wid * NODES_PER_W + c * NODE_CHUNK, 8)
        pltpu.sync_copy(start_hbm.at[pl.ds(n0, 1608)], sv)

        def build(j, _):
            lo = sv[pl.ds(j * L, L)]
            hi = sv[pl.ds(j * L + 1, L)]
            eidx[pl.ds(j * L, L)] = jnp.maximum(hi - 1, 0)
            bidx[pl.ds(j * L, L)] = jnp.maximum(lo - 1, 0)
            return 0

        lax.fori_loop(0, NODE_CHUNK // L, build, 0)
        pltpu.async_copy(cums_hbm.at[eidx], le_v, sem).wait()
        pltpu.async_copy(cums_hbm.at[bidx], lb_v, sem).wait()

        def comp(j, _):
            lo = sv[pl.ds(j * L, L)]
            hi = sv[pl.ds(j * L + 1, L)]
            e = hi - 1
            b = lo - 1
            ec = jnp.maximum(e, 0)
            bc = jnp.maximum(b, 0)
            be = plsc.load_gather(bases_v, [ec // SLOTS_PER_W])
            bb = plsc.load_gather(bases_v, [bc // SLOTS_PER_W])
            ge = jnp.where(e < 0, jnp.float32(0.0), le_v[pl.ds(j * L, L)] + be)
            gb = jnp.where(b < 0, jnp.float32(0.0), lb_v[pl.ds(j * L, L)] + bb)
            ov[pl.ds(j * L, L)] = ge - gb
            return 0

        lax.fori_loop(0, NODE_CHUNK // L, comp, 0)
        pltpu.sync_copy(ov, out_hbm.at[pl.ds(n0, NODE_CHUNK)])
        return 0

    lax.fori_loop(0, NODES_PER_W // NODE_CHUNK, chunk_body, 0)


def kernel(net_weights, flat_nodepin, nodepin_start, pin2net_map, num_nodes):
    n = nodepin_start.shape[0] - 1
    p = flat_nodepin.shape[0]
    fnp = flat_nodepin.astype(jnp.int32)
    p2n = pin2net_map.astype(jnp.int32)
    start_pad = jnp.concatenate([
        nodepin_start.astype(jnp.int32),
        jnp.full((SPAD_LEN - (n + 1),), p, dtype=jnp.int32),
    ])
    cums, totals = _phase1(net_weights, fnp, p2n)
    out_pad = _phase2(cums, totals[:, 0], start_pad)
    return out_pad[:n]
